# baseline (speedup 1.0000x reference)
import functools

import jax
import jax.numpy as jnp
from jax import lax
from jax.experimental import pallas as pl
from jax.experimental.pallas import tpu as pltpu
from jax.experimental.pallas import tpu_sc as plsc

NC = 2
NS = 16
NW = NC * NS
CHUNK = 128
D = 128
NB = 5
LOOKAHEAD = 3


@functools.cache
def _emb_kernel(n_idx: int):
  b_per_w = n_idx // NW
  n_chunks = b_per_w // CHUNK
  assert n_chunks % NB == 0
  mesh = plsc.VectorSubcoreMesh(
      core_axis_name="c", subcore_axis_name="s", num_cores=NC, num_subcores=NS
  )

  @functools.partial(
      pl.kernel,
      out_type=jax.ShapeDtypeStruct((n_idx, D), jnp.float32),
      mesh=mesh,
      scratch_types=[
          pltpu.VMEM((b_per_w,), jnp.int32),
          pltpu.VMEM((NB, CHUNK, D), jnp.float32),
          [pltpu.SemaphoreType.DMA for _ in range(NB)],
          [pltpu.SemaphoreType.DMA for _ in range(NB)],
      ],
  )
  def k(words_hbm, table_hbm, out_hbm, idx_v, rows_v, gsems, osems):
    wid = lax.axis_index("s") * NC + lax.axis_index("c")
    base = wid * b_per_w
    pltpu.sync_copy(words_hbm.at[pl.ds(base, b_per_w)], idx_v)

    def fire_gather(c, slot):
      pltpu.async_copy(
          table_hbm.at[idx_v.at[pl.ds(c * CHUNK, CHUNK)]],
          rows_v.at[slot],
          gsems[slot],
      )

    def wait_write(slot):
      pltpu.make_async_copy(
          table_hbm.at[pl.ds(0, CHUNK)], rows_v.at[slot], osems[slot]
      ).wait()

    for c in range(LOOKAHEAD):
      fire_gather(c, c % NB)

    def outer(o, _):
      for b in range(NB):
        j = o * NB + b
        pltpu.make_async_copy(
            table_hbm.at[pl.ds(0, CHUNK)], rows_v.at[b], gsems[b]
        ).wait()
        pltpu.async_copy(
            rows_v.at[b],
            out_hbm.at[pl.ds(base + j * CHUNK, CHUNK)],
            osems[b],
        )
        nb = (b + LOOKAHEAD) % NB

        @pl.when(j + LOOKAHEAD < n_chunks)
        def _():
          @pl.when(j + LOOKAHEAD >= NB)
          def _():
            wait_write(nb)

          fire_gather(j + LOOKAHEAD, nb)

      return 0

    lax.fori_loop(0, n_chunks // NB, outer, 0)

    for b in range(NB):
      wait_write(b)

  return k


def kernel(words, table):
  b, h = words.shape
  idx = words.reshape(-1).astype(jnp.int32)
  out = _emb_kernel(idx.shape[0])(idx, table.astype(jnp.float32))
  return out.reshape(b, h, D)


# speedup vs baseline: 3.1264x; 3.1264x over previous
import functools

import jax
import jax.numpy as jnp
from jax import lax
from jax.experimental import pallas as pl
from jax.experimental.pallas import tpu as pltpu
from jax.experimental.pallas import tpu_sc as plsc

NC = 2
NS = 16
NW = NC * NS
CHUNK = 128
D = 128
NB = 5
LOOKAHEAD = 3


@functools.cache
def _emb_kernel(n_idx: int):
  b_per_w = n_idx // NW
  n_chunks = b_per_w // CHUNK
  assert n_chunks % NB == 0
  mesh = plsc.VectorSubcoreMesh(
      core_axis_name="c", subcore_axis_name="s", num_cores=NC, num_subcores=NS
  )

  @functools.partial(
      pl.kernel,
      out_type=jax.ShapeDtypeStruct((n_idx, D), jnp.float32),
      mesh=mesh,
      scratch_types=[
          pltpu.VMEM((b_per_w,), jnp.int32),
          pltpu.VMEM((NB, CHUNK, D), jnp.float32),
          [pltpu.SemaphoreType.DMA for _ in range(NB)],
          [pltpu.SemaphoreType.DMA for _ in range(NB)],
      ],
  )
  def k(words_hbm, table_hbm, out_hbm, idx_v, rows_v, gsems, osems):
    wid = lax.axis_index("s") * NC + lax.axis_index("c")
    base = wid * b_per_w
    pltpu.sync_copy(words_hbm.at[pl.ds(base, b_per_w)], idx_v)

    def fire_gather(c, slot):
      pltpu.async_copy(
          table_hbm.at[idx_v.at[pl.ds(c * CHUNK, CHUNK)]],
          rows_v.at[slot],
          gsems[slot],
      )

    def wait_write(slot):
      pltpu.make_async_copy(
          table_hbm.at[pl.ds(0, CHUNK)], rows_v.at[slot], osems[slot]
      ).wait()

    for c in range(LOOKAHEAD):
      fire_gather(c, c % NB)

    def outer(o, _):
      for b in range(NB):
        j = o * NB + b
        pltpu.make_async_copy(
            table_hbm.at[pl.ds(0, CHUNK)], rows_v.at[b], gsems[b]
        ).wait()
        pltpu.async_copy(
            rows_v.at[b],
            out_hbm.at[pl.ds(base + j * CHUNK, CHUNK)],
            osems[b],
        )
        nb = (b + LOOKAHEAD) % NB

        @pl.when(j + LOOKAHEAD < n_chunks)
        def _():
          @pl.when(j + LOOKAHEAD >= NB)
          def _():
            wait_write(nb)

          fire_gather(j + LOOKAHEAD, nb)

      return 0

    lax.fori_loop(0, n_chunks // NB, outer, 0)

    for b in range(NB):
      wait_write(b)

  return k


def kernel(words, table):
  b, h = words.shape
  idx = words.T.reshape(-1).astype(jnp.int32)
  out = _emb_kernel(idx.shape[0])(idx, table.astype(jnp.float32))
  return out.reshape(h, b, D).transpose(1, 0, 2)


# speedup vs baseline: 3.1324x; 1.0019x over previous
import functools

import jax
import jax.numpy as jnp
from jax import lax
from jax.experimental import pallas as pl
from jax.experimental.pallas import tpu as pltpu
from jax.experimental.pallas import tpu_sc as plsc

NC = 2
NS = 16
NW = NC * NS
CHUNK = 128
D = 128
NB = 5
LOOKAHEAD = 4


@functools.cache
def _emb_kernel(n_idx: int):
  b_per_w = n_idx // NW
  n_chunks = b_per_w // CHUNK
  assert n_chunks % NB == 0
  mesh = plsc.VectorSubcoreMesh(
      core_axis_name="c", subcore_axis_name="s", num_cores=NC, num_subcores=NS
  )

  @functools.partial(
      pl.kernel,
      out_type=jax.ShapeDtypeStruct((n_idx, D), jnp.float32),
      mesh=mesh,
      scratch_types=[
          pltpu.VMEM((b_per_w,), jnp.int32),
          pltpu.VMEM((NB, CHUNK, D), jnp.float32),
          [pltpu.SemaphoreType.DMA for _ in range(NB)],
          [pltpu.SemaphoreType.DMA for _ in range(NB)],
      ],
  )
  def k(words_hbm, table_hbm, out_hbm, idx_v, rows_v, gsems, osems):
    wid = lax.axis_index("s") * NC + lax.axis_index("c")
    base = wid * b_per_w
    pltpu.sync_copy(words_hbm.at[pl.ds(base, b_per_w)], idx_v)

    def fire_gather(c, slot):
      pltpu.async_copy(
          table_hbm.at[idx_v.at[pl.ds(c * CHUNK, CHUNK)]],
          rows_v.at[slot],
          gsems[slot],
      )

    def wait_write(slot):
      pltpu.make_async_copy(
          table_hbm.at[pl.ds(0, CHUNK)], rows_v.at[slot], osems[slot]
      ).wait()

    for c in range(LOOKAHEAD):
      fire_gather(c, c % NB)

    def outer(o, _):
      for b in range(NB):
        j = o * NB + b
        pltpu.make_async_copy(
            table_hbm.at[pl.ds(0, CHUNK)], rows_v.at[b], gsems[b]
        ).wait()
        pltpu.async_copy(
            rows_v.at[b],
            out_hbm.at[pl.ds(base + j * CHUNK, CHUNK)],
            osems[b],
        )
        nb = (b + LOOKAHEAD) % NB

        @pl.when(j + LOOKAHEAD < n_chunks)
        def _():
          @pl.when(j + LOOKAHEAD >= NB)
          def _():
            wait_write(nb)

          fire_gather(j + LOOKAHEAD, nb)

      return 0

    lax.fori_loop(0, n_chunks // NB, outer, 0)

    for b in range(NB):
      wait_write(b)

  return k


def kernel(words, table):
  b, h = words.shape
  idx = words.T.reshape(-1).astype(jnp.int32)
  out = _emb_kernel(idx.shape[0])(idx, table.astype(jnp.float32))
  return out.reshape(h, b, D).transpose(1, 0, 2)
